# baseline (speedup 1.0000x reference)
import functools

import jax
import jax.numpy as jnp
from jax.experimental import pallas as pl
from jax.experimental.pallas import tpu as pltpu

N = 50000
E = 800000
HD = 64


def _leaky(x, s=0.01):
    return jnp.where(x > 0, x, s * x)


def _enc_kernel(des_ref, tw_ref, np_ref, cp_ref,
                wd, bd, wt, bt, wn, bn, wc, bc, wi, bi,
                wd2, bd2, wt2, bt2, wn2, bn2, wc2, bc2, wi2, bi2,
                x_ref, xg_ref):
    des = des_ref[...]
    tw = tw_ref[...]
    npb = np_ref[...]
    cpb = cp_ref[...]

    def enc(wdr, bdr, wtr, btr, wnr, bnr, wcr, bcr, wir, bir):
        d = _leaky(jnp.dot(des, wdr[...], preferred_element_type=jnp.float32) + bdr[...])
        t = _leaky(jnp.dot(tw, wtr[...], preferred_element_type=jnp.float32) + btr[...])
        n = _leaky(jnp.dot(npb, wnr[...], preferred_element_type=jnp.float32) + bnr[...])
        c = _leaky(jnp.dot(cpb, wcr[...], preferred_element_type=jnp.float32) + bcr[...])
        h = jnp.concatenate([d, t, n, c], axis=1)
        return _leaky(jnp.dot(h, wir[...], preferred_element_type=jnp.float32) + bir[...])

    x_ref[...] = enc(wd, bd, wt, bt, wn, bn, wc, bc, wi, bi)
    xg_ref[...] = enc(wd2, bd2, wt2, bt2, wn2, bn2, wc2, bc2, wi2, bi2)


def _encode_both(des, tweet, num_prop, cat_prop,
                 wd, bd, wt, bt, wn, bn, wc, bc, wi, bi,
                 wd2, bd2, wt2, bt2, wn2, bn2, wc2, bc2, wi2, bi2):
    BM = 2000
    q = HD // 4
    full = lambda shape: pl.BlockSpec(shape, lambda i: (0, 0))
    row = lambda shape: pl.BlockSpec(shape, lambda i: (i, 0))
    return pl.pallas_call(
        _enc_kernel,
        out_shape=(jax.ShapeDtypeStruct((N, HD), jnp.float32),
                   jax.ShapeDtypeStruct((N, HD), jnp.float32)),
        grid=(N // BM,),
        in_specs=[
            row((BM, 768)), row((BM, 768)), row((BM, 5)), row((BM, 3)),
            full((768, q)), full((1, q)), full((768, q)), full((1, q)),
            full((5, q)), full((1, q)), full((3, q)), full((1, q)),
            full((HD, HD)), full((1, HD)),
            full((768, q)), full((1, q)), full((768, q)), full((1, q)),
            full((5, q)), full((1, q)), full((3, q)), full((1, q)),
            full((HD, HD)), full((1, HD)),
        ],
        out_specs=(row((BM, HD)), row((BM, HD))),
    )(des, tweet, num_prop, cat_prop,
      wd, bd.reshape(1, -1), wt, bt.reshape(1, -1),
      wn, bn.reshape(1, -1), wc, bc.reshape(1, -1),
      wi, bi.reshape(1, -1),
      wd2, bd2.reshape(1, -1), wt2, bt2.reshape(1, -1),
      wn2, bn2.reshape(1, -1), wc2, bc2.reshape(1, -1),
      wi2, bi2.reshape(1, -1))


def _gat_conv(x, src, dst, W, att_s, att_d, bias, heads, out_ch):
    h = (x @ W).reshape(N, heads, out_ch)
    a_s = (h * att_s[None]).sum(-1)
    a_d = (h * att_d[None]).sum(-1)
    alpha = a_s[src] + a_d[dst]
    alpha = jnp.where(alpha > 0, alpha, 0.2 * alpha)
    amax = jax.ops.segment_max(alpha, dst, num_segments=N)
    amax = jnp.where(jnp.isfinite(amax), amax, 0.0)
    e = jnp.exp(alpha - amax[dst])
    den = jax.ops.segment_sum(e, dst, num_segments=N)
    coef = e / (den[dst] + 1e-16)
    msg = h[src] * coef[:, :, None]
    out = jax.ops.segment_sum(msg, dst, num_segments=N)
    return out.reshape(N, heads * out_ch) + bias


def _gcn_conv(x, src, dst, W, bias):
    deg = jax.ops.segment_sum(jnp.ones(src.shape[0], jnp.float32), dst, num_segments=N)
    dinv = jnp.where(deg > 0, deg ** -0.5, 0.0)
    norm = dinv[src] * dinv[dst]
    h = x @ W
    out = jax.ops.segment_sum(h[src] * norm[:, None], dst, num_segments=N)
    return out + bias


def _final_matmul_kernel(x_ref, w_ref, b_ref, o_ref):
    o_ref[...] = x_ref[...] @ w_ref[...] + b_ref[...]


def _final_matmul(stack, we, be):
    M = stack.shape[0]
    BM = 2000
    return pl.pallas_call(
        _final_matmul_kernel,
        out_shape=jax.ShapeDtypeStruct((M, 2), jnp.float32),
        grid=(M // BM,),
        in_specs=[
            pl.BlockSpec((BM, HD), lambda i: (i, 0)),
            pl.BlockSpec((HD, 2), lambda i: (0, 0)),
            pl.BlockSpec((1, 2), lambda i: (0, 0)),
        ],
        out_specs=pl.BlockSpec((BM, 2), lambda i: (i, 0)),
    )(stack, we, be.reshape(1, 2))


def kernel(des, tweet, num_prop, cat_prop, edge_index, wd, bd, wt, bt, wn, bn, wc, bc, wi, bi, g1w, g1as, g1ad, g1b, g2w, g2as, g2ad, g2b, wo, bo, wd2, bd2, wt2, bt2, wn2, bn2, wc2, bc2, wi2, bi2, c1w, c1b, c2w, c2b, wo2, bo2, we, be):
    loops = jnp.arange(N, dtype=edge_index.dtype)
    src = jnp.concatenate([edge_index[0], loops])
    dst = jnp.concatenate([edge_index[1], loops])
    x, xg = _encode_both(des, tweet, num_prop, cat_prop,
                         wd, bd, wt, bt, wn, bn, wc, bc, wi, bi,
                         wd2, bd2, wt2, bt2, wn2, bn2, wc2, bc2, wi2, bi2)
    x = _gat_conv(x, src, dst, g1w, g1as, g1ad, g1b, 4, HD // 4)
    x = _gat_conv(x, src, dst, g2w, g2as, g2ad, g2b, 1, HD)
    x = _leaky(x @ wo + bo)
    xg = _gcn_conv(xg, src, dst, c1w, c1b)
    xg = _gcn_conv(xg, src, dst, c2w, c2b)
    xg = _leaky(xg @ wo2 + bo2)
    stack = jnp.concatenate([x, xg], axis=0)
    return _final_matmul(stack, we, be)
